# baseline (speedup 1.0000x reference)
import functools

import jax
import jax.numpy as jnp
from jax import lax
from jax.experimental import pallas as pl
from jax.experimental.pallas import tpu as pltpu
from jax.experimental.pallas import tpu_sc as plsc

_LANES = 16
_BLK_R = 8


def _sc_body(h_vmem, t_vmem, o_vmem):
    rows, hidden = t_vmem.shape

    group = 8
    for r in range(rows):
        for c0 in range(0, hidden, group * _LANES):
            sls = [pl.ds(c0 + k * _LANES, _LANES) for k in range(group)]
            ts = [t_vmem[r, sl] for sl in sls]
            h0s = [h_vmem[0, r, sl] for sl in sls]
            h1s = [h_vmem[1, r, sl] for sl in sls]
            for k, sl in enumerate(sls):
                o_vmem[0, r, sl] = h0s[k] + ts[k]
            for k, sl in enumerate(sls):
                o_vmem[1, r, sl] = h1s[k] + ts[k]


def _sc_kernel_body(h_hbm, t_hbm, o_hbm):
    batch, seq, hidden = h_hbm.shape
    grid = (seq // _BLK_R,)
    pltpu.emit_pipeline(
        _sc_body,
        grid=grid,
        in_specs=[
            pl.BlockSpec((batch, _BLK_R, hidden), lambda i: (0, i, 0)),
            pl.BlockSpec((_BLK_R, hidden), lambda i: (i, 0)),
        ],
        out_specs=[pl.BlockSpec((batch, _BLK_R, hidden), lambda i: (0, i, 0))],
        core_axis_name=("core", "subcore"),
        dimension_semantics=(pltpu.PARALLEL,),
    )(h_hbm, t_hbm, o_hbm)


def kernel(hidden_states, table, seq_len):
    batch, static_len, hidden = hidden_states.shape
    off = seq_len - static_len
    table = lax.dynamic_slice(table, (off, 0), (static_len, hidden))

    mesh = plsc.VectorSubcoreMesh(core_axis_name="core", subcore_axis_name="subcore")
    run = pl.kernel(
        _sc_kernel_body,
        out_type=jax.ShapeDtypeStruct(hidden_states.shape, hidden_states.dtype),
        mesh=mesh,
    )
    return run(hidden_states, table)
